# baseline (speedup 1.0000x reference)
import functools

import jax
import jax.numpy as jnp
from jax import lax
from jax.experimental import pallas as pl
from jax.experimental.pallas import tpu as pltpu
from jax.experimental.pallas import tpu_sc as plsc

NB = 4096
PAIRS = 36
NPG = 9
N = NB * NPG
E = NB * PAIRS * 2
H = 64
L = 4

NC = 2
NS = 16
NW = NC * NS
NHALF = N // NC
NPAD = 18944


def _make_gather(n_idx, total, d, ch):
    pw = total // NW
    steps = pw // ch
    assert pw % ch == 0 and ch % 8 == 0

    def body(*refs):
        table = refs[0]
        idxs = refs[1:1 + n_idx]
        outs = refs[1 + n_idx:1 + 2 * n_idx]
        idx_v, rows_v, sem = refs[1 + 2 * n_idx:]
        wid = lax.axis_index("s") * NC + lax.axis_index("c")
        base = wid * pw

        def step(i, carry):
            off = base + i * ch
            for k in range(n_idx):
                pltpu.sync_copy(idxs[k].at[pl.ds(off, ch)], idx_v)
                pltpu.async_copy(table.at[idx_v], rows_v, sem).wait()
                pltpu.sync_copy(rows_v, outs[k].at[pl.ds(off, ch)])
            return carry

        lax.fori_loop(0, steps, step, 0)

    out_type = [jax.ShapeDtypeStruct((total, d), jnp.float32)] * n_idx
    return pl.kernel(
        body,
        out_type=out_type,
        mesh=_mesh(),
        scratch_types=[
            pltpu.VMEM((ch,), jnp.int32),
            pltpu.VMEM((ch, d), jnp.float32),
            pltpu.SemaphoreType.DMA,
        ],
        compiler_params=pltpu.CompilerParams(use_tc_tiling_on_sc=False),
    )


def _fill_rows(ref, rows, d, value):
    def step(i, carry):
        for j in range(d // 16):
            ref[i, pl.ds(j * 16, 16)] = jnp.full((16,), value, jnp.float32)
        return carry
    lax.fori_loop(0, rows, step, 0)


def _make_scatter_add(d, read_rows, ch):
    es = E // NS
    steps = es // ch
    assert es % ch == 0 and ch % 16 == 0
    zr = 32
    per_sub_pad = NPAD // NS
    per_sub_out = NHALF // NS

    def body(*refs):
        if read_rows:
            colidx, grow, out, idx_v, rows_v, zbuf, acc, sem = refs
        else:
            colidx, out, idx_v, rows_v, zbuf, acc, sem = refs
        c = lax.axis_index("c")
        s = lax.axis_index("s")
        nbase = c * NHALF

        _fill_rows(zbuf, zr, d, 0.0)
        if not read_rows:
            _fill_rows(rows_v, ch, d, 1.0)
        for t in range(per_sub_pad // zr):
            pltpu.sync_copy(zbuf, acc.at[pl.ds(s * per_sub_pad + t * zr, zr)])
        plsc.subcore_barrier()

        def step(i, carry):
            off = s * es + i * ch
            pltpu.sync_copy(colidx.at[pl.ds(off, ch)], idx_v)
            if read_rows:
                pltpu.sync_copy(grow.at[pl.ds(off, ch)], rows_v)

            def remap(j, carry2):
                v = idx_v[pl.ds(j * 16, 16)] - nbase
                ok = (v >= 0) & (v < NHALF)
                idx_v[pl.ds(j * 16, 16)] = jnp.where(ok, v, NHALF)
                return carry2

            lax.fori_loop(0, ch // 16, remap, 0)
            pltpu.sync_copy(rows_v, acc.at[idx_v], add=True)
            return carry

        lax.fori_loop(0, steps, step, 0)
        plsc.subcore_barrier()
        pltpu.sync_copy(
            acc.at[pl.ds(s * per_sub_out, per_sub_out)],
            out.at[pl.ds(nbase + s * per_sub_out, per_sub_out)],
        )

    return pl.kernel(
        body,
        out_type=jax.ShapeDtypeStruct((N, d), jnp.float32),
        mesh=_mesh(),
        scratch_types=[
            pltpu.VMEM((ch,), jnp.int32),
            pltpu.VMEM((ch, d), jnp.float32),
            pltpu.VMEM((zr, d), jnp.float32),
            pltpu.VMEM_SHARED((NPAD, d), jnp.float32),
            pltpu.SemaphoreType.DMA,
        ],
        compiler_params=pltpu.CompilerParams(use_tc_tiling_on_sc=False),
    )


def _make_layer_sc(chp):
    p2 = (E // 2) // NC
    po = p2 // NS
    steps = po // chp
    assert po % chp == 0 and chp % 16 == 0
    zr = 32
    per_sub_pad = NPAD // NS
    per_sub_out = NHALF // NS

    assert steps % 2 == 0

    def body(hn, row_e, row_o, col_e, col_o, gr2, gc2, out_aggr, *scr):
        idxb = [scr[0:4], scr[4:8]]
        rowb = [scr[8:12], scr[12:16]]
        zbuf = scr[16]
        acc = scr[17]
        sem_g = scr[18:20]
        sem_w = scr[20:22]
        sem_s = scr[22:24]
        c = lax.axis_index("c")
        s = lax.axis_index("s")
        nbase = c * NHALF

        _fill_rows(zbuf, zr, H, 0.0)
        for t in range(per_sub_pad // zr):
            pltpu.sync_copy(zbuf, acc.at[pl.ds(s * per_sub_pad + t * zr, zr)])
        plsc.subcore_barrier()

        own_base = c * p2 + s * po
        for_base = (p2 - c * p2) + s * po

        def remap(idx_ref):
            def rstep(j, carry):
                v = idx_ref[pl.ds(j * 16, 16)] - nbase
                ok = (v >= 0) & (v < NHALF)
                idx_ref[pl.ds(j * 16, 16)] = jnp.where(ok, v, NHALF)
                return carry
            lax.fori_loop(0, chp // 16, rstep, 0)

        def issue_own(off, p):
            i_re, i_ro, i_ce, i_co = idxb[p]
            r_re, r_ro, r_ce, r_co = rowb[p]
            pltpu.sync_copy(row_e.at[pl.ds(off, chp)], i_re)
            pltpu.async_copy(hn.at[i_re], r_re, sem_g[p])
            pltpu.sync_copy(row_o.at[pl.ds(off, chp)], i_ro)
            pltpu.async_copy(hn.at[i_ro], r_ro, sem_g[p])
            pltpu.sync_copy(col_e.at[pl.ds(off, chp)], i_ce)
            pltpu.async_copy(hn.at[i_ce], r_ce, sem_g[p])
            pltpu.sync_copy(col_o.at[pl.ds(off, chp)], i_co)
            pltpu.async_copy(hn.at[i_co], r_co, sem_g[p])

        def retire_own(off, p):
            i_re, i_ro, i_ce, i_co = idxb[p]
            r_re, r_ro, r_ce, r_co = rowb[p]
            for r in (r_re, r_ro, r_ce, r_co):
                pltpu.make_async_copy(hn.at[i_re], r, sem_g[p]).wait()
            pltpu.async_copy(r_re, gr2.at[pl.ds(off, chp), pl.ds(0, H)],
                             sem_w[p])
            pltpu.async_copy(r_ro, gr2.at[pl.ds(off, chp), pl.ds(H, H)],
                             sem_w[p])
            pltpu.async_copy(r_ce, gc2.at[pl.ds(off, chp), pl.ds(0, H)],
                             sem_w[p])
            pltpu.async_copy(r_co, gc2.at[pl.ds(off, chp), pl.ds(H, H)],
                             sem_w[p])
            remap(i_ce)
            remap(i_co)
            pltpu.async_copy(r_re, acc.at[i_ce], sem_s[p], add=True)
            pltpu.async_copy(r_ro, acc.at[i_co], sem_s[p], add=True)

        def drain_own(p):
            i_re, i_ro, i_ce, i_co = idxb[p]
            r_re, r_ro, r_ce, r_co = rowb[p]
            for r in (r_re, r_ro, r_ce, r_co):
                pltpu.make_async_copy(
                    r, gr2.at[pl.ds(own_base, chp), pl.ds(0, H)],
                    sem_w[p]).wait()
            pltpu.make_async_copy(r_re, acc.at[i_ce], sem_s[p]).wait()
            pltpu.make_async_copy(r_ro, acc.at[i_co], sem_s[p]).wait()

        issue_own(own_base, 0)

        def own_iter2(i2, carry):
            a = 2 * i2

            @pl.when(i2 > 0)
            def _():
                drain_own(0)
                issue_own(own_base + a * chp, 0)
                retire_own(own_base + (a - 1) * chp, 1)

            @pl.when(i2 > 0)
            def _():
                drain_own(1)
            issue_own(own_base + (a + 1) * chp, 1)
            retire_own(own_base + a * chp, 0)
            return carry

        lax.fori_loop(0, steps // 2, own_iter2, 0)
        retire_own(own_base + (steps - 1) * chp, 1)
        drain_own(0)
        drain_own(1)

        def issue_f(off, p):
            i_re, i_ro, i_ce, i_co = idxb[p]
            r_re, r_ro = rowb[p][0], rowb[p][1]
            pltpu.sync_copy(row_e.at[pl.ds(off, chp)], i_re)
            pltpu.async_copy(hn.at[i_re], r_re, sem_g[p])
            pltpu.sync_copy(row_o.at[pl.ds(off, chp)], i_ro)
            pltpu.async_copy(hn.at[i_ro], r_ro, sem_g[p])
            pltpu.sync_copy(col_e.at[pl.ds(off, chp)], i_ce)
            pltpu.sync_copy(col_o.at[pl.ds(off, chp)], i_co)

        def retire_f(p):
            i_re, i_ro, i_ce, i_co = idxb[p]
            r_re, r_ro = rowb[p][0], rowb[p][1]
            for r in (r_re, r_ro):
                pltpu.make_async_copy(hn.at[i_re], r, sem_g[p]).wait()
            remap(i_ce)
            remap(i_co)
            pltpu.async_copy(r_re, acc.at[i_ce], sem_s[p], add=True)
            pltpu.async_copy(r_ro, acc.at[i_co], sem_s[p], add=True)

        def drain_f(p):
            i_re, i_ro, i_ce, i_co = idxb[p]
            r_re, r_ro = rowb[p][0], rowb[p][1]
            pltpu.make_async_copy(r_re, acc.at[i_ce], sem_s[p]).wait()
            pltpu.make_async_copy(r_ro, acc.at[i_co], sem_s[p]).wait()

        issue_f(for_base, 0)

        def f_iter(i2, carry):
            a = 2 * i2

            @pl.when(i2 > 0)
            def _():
                drain_f(0)
                issue_f(for_base + a * chp, 0)
                retire_f(1)

            @pl.when(i2 > 0)
            def _():
                drain_f(1)
            issue_f(for_base + (a + 1) * chp, 1)
            retire_f(0)
            return carry

        lax.fori_loop(0, steps // 2, f_iter, 0)
        retire_f(1)
        drain_f(0)
        drain_f(1)

        plsc.subcore_barrier()
        pltpu.sync_copy(
            acc.at[pl.ds(s * per_sub_out, per_sub_out)],
            out_aggr.at[pl.ds(nbase + s * per_sub_out, per_sub_out)],
        )

    return pl.kernel(
        body,
        out_type=[
            jax.ShapeDtypeStruct((E // 2, 2 * H), jnp.float32),
            jax.ShapeDtypeStruct((E // 2, 2 * H), jnp.float32),
            jax.ShapeDtypeStruct((N, H), jnp.float32),
        ],
        mesh=_mesh(),
        scratch_types=(
            [pltpu.VMEM((chp,), jnp.int32)] * 8
            + [pltpu.VMEM((chp, H), jnp.float32)] * 8
            + [pltpu.VMEM((zr, H), jnp.float32),
               pltpu.VMEM_SHARED((NPAD, H), jnp.float32)]
            + [pltpu.SemaphoreType.DMA] * 6
        ),
        compiler_params=pltpu.CompilerParams(use_tc_tiling_on_sc=False),
    )


def _make_gxd_packed(chp):
    pw = (E // 2) // NW
    steps = pw // chp
    assert pw % chp == 0 and chp % 8 == 0
    d = 16

    def body(xd, row_e, row_o, col_e, col_o, out,
             i_re, i_ro, i_ce, i_co, r1, r2, r3, r4,
             sem1, sem2, sem3, sem4, sem_w):
        wid = lax.axis_index("s") * NC + lax.axis_index("c")
        base = wid * pw

        def step(i, carry):
            off = base + i * chp
            pltpu.sync_copy(row_e.at[pl.ds(off, chp)], i_re)
            g1 = pltpu.async_copy(xd.at[i_re], r1, sem1)
            pltpu.sync_copy(row_o.at[pl.ds(off, chp)], i_ro)
            g2 = pltpu.async_copy(xd.at[i_ro], r2, sem2)
            pltpu.sync_copy(col_e.at[pl.ds(off, chp)], i_ce)
            g3 = pltpu.async_copy(xd.at[i_ce], r3, sem3)
            pltpu.sync_copy(col_o.at[pl.ds(off, chp)], i_co)
            g4 = pltpu.async_copy(xd.at[i_co], r4, sem4)
            g1.wait()
            w1 = pltpu.async_copy(
                r1, out.at[pl.ds(off, chp), pl.ds(0, d)], sem_w)
            g2.wait()
            w2 = pltpu.async_copy(
                r2, out.at[pl.ds(off, chp), pl.ds(d, d)], sem_w)
            g3.wait()
            w3 = pltpu.async_copy(
                r3, out.at[pl.ds(off, chp), pl.ds(2 * d, d)], sem_w)
            g4.wait()
            w4 = pltpu.async_copy(
                r4, out.at[pl.ds(off, chp), pl.ds(3 * d, d)], sem_w)
            w1.wait()
            w2.wait()
            w3.wait()
            w4.wait()
            return carry

        lax.fori_loop(0, steps, step, 0)

    return pl.kernel(
        body,
        out_type=jax.ShapeDtypeStruct((E // 2, 2 * H), jnp.float32),
        mesh=_mesh(),
        scratch_types=(
            [pltpu.VMEM((chp,), jnp.int32)] * 4
            + [pltpu.VMEM((chp, d), jnp.float32)] * 4
            + [pltpu.SemaphoreType.DMA] * 5
        ),
        compiler_params=pltpu.CompilerParams(use_tc_tiling_on_sc=False),
    )


def _make_gpair(chp):
    pw = (E // 2) // NW
    steps = pw // chp
    assert pw % chp == 0 and chp % 8 == 0

    def body(hn, row_e, row_o, col_e, col_o, gr2, gc2,
             i_re, i_ro, i_ce, i_co, rows_re, rows_ro, rows_ce, rows_co,
             sem1, sem2, sem3, sem4, sem_w):
        wid = lax.axis_index("s") * NC + lax.axis_index("c")
        base = wid * pw

        def step(i, carry):
            off = base + i * chp
            pltpu.sync_copy(row_e.at[pl.ds(off, chp)], i_re)
            g1 = pltpu.async_copy(hn.at[i_re], rows_re, sem1)
            pltpu.sync_copy(row_o.at[pl.ds(off, chp)], i_ro)
            g2 = pltpu.async_copy(hn.at[i_ro], rows_ro, sem2)
            pltpu.sync_copy(col_e.at[pl.ds(off, chp)], i_ce)
            g3 = pltpu.async_copy(hn.at[i_ce], rows_ce, sem3)
            pltpu.sync_copy(col_o.at[pl.ds(off, chp)], i_co)
            g4 = pltpu.async_copy(hn.at[i_co], rows_co, sem4)
            g1.wait()
            w1 = pltpu.async_copy(
                rows_re, gr2.at[pl.ds(off, chp), pl.ds(0, H)], sem_w)
            g2.wait()
            w2 = pltpu.async_copy(
                rows_ro, gr2.at[pl.ds(off, chp), pl.ds(H, H)], sem_w)
            g3.wait()
            w3 = pltpu.async_copy(
                rows_ce, gc2.at[pl.ds(off, chp), pl.ds(0, H)], sem_w)
            g4.wait()
            w4 = pltpu.async_copy(
                rows_co, gc2.at[pl.ds(off, chp), pl.ds(H, H)], sem_w)
            w1.wait()
            w2.wait()
            w3.wait()
            w4.wait()
            return carry

        lax.fori_loop(0, steps, step, 0)

    return pl.kernel(
        body,
        out_type=[
            jax.ShapeDtypeStruct((E // 2, 2 * H), jnp.float32),
            jax.ShapeDtypeStruct((E // 2, 2 * H), jnp.float32),
        ],
        mesh=_mesh(),
        scratch_types=[
            pltpu.VMEM((chp,), jnp.int32),
            pltpu.VMEM((chp,), jnp.int32),
            pltpu.VMEM((chp,), jnp.int32),
            pltpu.VMEM((chp,), jnp.int32),
            pltpu.VMEM((chp, H), jnp.float32),
            pltpu.VMEM((chp, H), jnp.float32),
            pltpu.VMEM((chp, H), jnp.float32),
            pltpu.VMEM((chp, H), jnp.float32),
            pltpu.SemaphoreType.DMA,
            pltpu.SemaphoreType.DMA,
            pltpu.SemaphoreType.DMA,
            pltpu.SemaphoreType.DMA,
            pltpu.SemaphoreType.DMA,
        ],
        compiler_params=pltpu.CompilerParams(use_tc_tiling_on_sc=False),
    )


@functools.cache
def _mesh():
    return plsc.VectorSubcoreMesh(core_axis_name="c", subcore_axis_name="s",
                                  num_cores=NC, num_subcores=NS)


@functools.cache
def _sc_kernels():
    return {
        "gdmg": _make_gather(1, N, 16, 1152),
        "gxd": _make_gxd_packed(512),
        "cnt": _make_scatter_add(16, False, 512),
        "layer": _make_layer_sc(96),
        "gpair": _make_gpair(288),
    }


BEP = 1024
BN = 2048
GE_ = (E // 2) // BEP
GN_ = N // BN


def _full(shape):
    return pl.BlockSpec(shape, lambda i: tuple(0 for _ in shape))


def _rows(blk, d):
    return pl.BlockSpec((blk, d), lambda i: (i, 0))


def _espec(d):
    return pl.BlockSpec((BEP, d), lambda i: (jnp.minimum(i, GE_ - 1), 0))


def _nspec(d):
    return pl.BlockSpec(
        (BN, d), lambda i: (jnp.clip(i - GE_, 0, GN_ - 1), 0))


def _edge_init_math(xdr, xdc, w1, b1, w2, b2):
    eps = 1e-8
    a = jnp.transpose(xdr)
    c = jnp.transpose(xdc)
    sx0, sx1, dg0, dg1 = a[0:1, :], a[1:2, :], a[2:3, :], a[3:4, :]
    dx0, dx1 = c[0:1, :], c[1:2, :]
    v0 = sx0 - dx0
    v1 = sx1 - dx1
    l2r = v0 * v0 + v1 * v1
    el = jnp.sqrt(l2r + eps)
    l2 = jnp.maximum(l2r, eps)
    t = jnp.clip(((dg0 - sx0) * (dx0 - sx0) + (dg1 - sx1) * (dx1 - sx1)) / l2,
                 0.0, 1.0)
    p0 = sx0 + t * (dx0 - sx0)
    p1 = sx1 + t * (dx1 - sx1)
    dfd = jnp.sqrt((dg0 - p0) ** 2 + (dg1 - p1) ** 2 + eps)
    dtx = jnp.sqrt((sx0 - dg0) ** 2 + (sx1 - dg1) ** 2 + eps)
    drx = jnp.sqrt((dx0 - dg0) ** 2 + (dx1 - dg1) ** 2 + eps)
    phys_t = jnp.concatenate([v0, v1, el, dfd, dtx, drx], axis=0)
    pre = lax.dot_general(phys_t, w1[...], (((0,), (0,)), ((), ())),
                          preferred_element_type=jnp.float32) + b1[...]
    hid = jnp.maximum(pre, 0.0)
    return jnp.dot(hid, w2[...], preferred_element_type=jnp.float32) + b2[...]


def _init_body(xdp, xd, w1, b1, w2, b2, wn, bn, out_he2, out_hn):
    pid = pl.program_id(0)

    @pl.when(pid < GE_)
    def _():
        he_e = _edge_init_math(xdp[:, 0:16], xdp[:, 32:48], w1, b1, w2, b2)
        he_o = _edge_init_math(xdp[:, 16:32], xdp[:, 48:64], w1, b1, w2, b2)
        out_he2[...] = jnp.concatenate([he_e, he_o], axis=1)

    @pl.when(pid >= GE_)
    def _():
        out_hn[...] = (xd[:, 0:1] * wn[0:1, :] + xd[:, 1:2] * wn[1:2, :]
                       + bn[...])


def _tc_init(xdp, xd, w1, b1, w2, b2, wn, bn):
    return pl.pallas_call(
        _init_body,
        grid=(GE_ + GN_,),
        in_specs=[
            _espec(2 * H), _nspec(16),
            _full((6, H)), _full((1, H)), _full((H, H)), _full((1, H)),
            _full((2, H)), _full((1, H)),
        ],
        out_specs=[_espec(2 * H), _nspec(H)],
        out_shape=[jax.ShapeDtypeStruct((E // 2, 2 * H), jnp.float32),
                   jax.ShapeDtypeStruct((N, H), jnp.float32)],
    )(xdp, xd, w1, b1, w2, b2, wn, bn)


def _layer_body(gr2, gc2, he2, hn, ag, cnt, w1a, w1b, w1c, b1, w2, b2,
                nw1a, nw1b, nb1, nw2, nb2, out_he2, out_hn):
    pid = pl.program_id(0)

    @pl.when(pid < GE_)
    def _():
        def upd(g_r, g_c, h):
            pre = (jnp.dot(g_r, w1a[...], preferred_element_type=jnp.float32)
                   + jnp.dot(g_c, w1b[...], preferred_element_type=jnp.float32)
                   + jnp.dot(h, w1c[...], preferred_element_type=jnp.float32)
                   + b1[...])
            hid = jnp.maximum(pre, 0.0)
            return h + jnp.dot(hid, w2[...],
                               preferred_element_type=jnp.float32) + b2[...]

        even = upd(gr2[:, 0:H], gc2[:, 0:H], he2[:, 0:H])
        odd = upd(gr2[:, H:2 * H], gc2[:, H:2 * H], he2[:, H:2 * H])
        out_he2[...] = jnp.concatenate([even, odd], axis=1)

    @pl.when(pid >= GE_)
    def _():
        aggr = ag[...] / jnp.maximum(cnt[:, 0:1], 1.0)
        pre = (jnp.dot(hn[...], nw1a[...], preferred_element_type=jnp.float32)
               + jnp.dot(aggr, nw1b[...], preferred_element_type=jnp.float32)
               + nb1[...])
        hid = jnp.maximum(pre, 0.0)
        out_hn[...] = hn[...] + jnp.dot(
            hid, nw2[...], preferred_element_type=jnp.float32) + nb2[...]


def _final_body(gr2, gc2, he2, w1a, w1b, w1c, b1, w2, b2,
                wd1, bd1, wd2t, bd2, out_pred):
    def upd(g_r, g_c, h):
        pre = (jnp.dot(g_r, w1a[...], preferred_element_type=jnp.float32)
               + jnp.dot(g_c, w1b[...], preferred_element_type=jnp.float32)
               + jnp.dot(h, w1c[...], preferred_element_type=jnp.float32)
               + b1[...])
        hid = jnp.maximum(pre, 0.0)
        return h + jnp.dot(hid, w2[...],
                           preferred_element_type=jnp.float32) + b2[...]

    def head(x):
        hid = jnp.maximum(
            jnp.dot(x, wd1[...], preferred_element_type=jnp.float32)
            + bd1[...], 0.0)
        logit = jnp.sum(hid * wd2t[...], axis=1, keepdims=True) + bd2[...]
        return 1.0 / (1.0 + jnp.exp(-logit))

    even = upd(gr2[:, 0:H], gc2[:, 0:H], he2[:, 0:H])
    odd = upd(gr2[:, H:2 * H], gc2[:, H:2 * H], he2[:, H:2 * H])
    out_pred[...] = 0.5 * (head(even) + head(odd))


def _tc_final(gr2, gc2, he2, w1a, w1b, w1c, b1, w2, b2, wd1, bd1, wd2t, bd2):
    return pl.pallas_call(
        _final_body,
        grid=(GE_,),
        in_specs=[
            _rows(BEP, 2 * H), _rows(BEP, 2 * H), _rows(BEP, 2 * H),
            _full((H, H)), _full((H, H)), _full((H, H)), _full((1, H)),
            _full((H, H)), _full((1, H)),
            _full((H, H // 2)), _full((1, H // 2)), _full((1, H // 2)),
            _full((1, 1)),
        ],
        out_specs=_rows(BEP, 1),
        out_shape=jax.ShapeDtypeStruct((E // 2, 1), jnp.float32),
    )(gr2, gc2, he2, w1a, w1b, w1c, b1, w2, b2, wd1, bd1, wd2t, bd2)


def _tc_layer(gr2, gc2, he2, hn, ag, cnt, w1a, w1b, w1c, b1, w2, b2,
              nw1a, nw1b, nb1, nw2, nb2):
    return pl.pallas_call(
        _layer_body,
        grid=(GE_ + GN_,),
        in_specs=[
            _espec(2 * H), _espec(2 * H), _espec(2 * H),
            _nspec(H), _nspec(H), _nspec(16),
            _full((H, H)), _full((H, H)), _full((H, H)), _full((1, H)),
            _full((H, H)), _full((1, H)),
            _full((H, H)), _full((H, H)), _full((1, H)),
            _full((H, H)), _full((1, H)),
        ],
        out_specs=[_espec(2 * H), _nspec(H)],
        out_shape=[jax.ShapeDtypeStruct((E // 2, 2 * H), jnp.float32),
                   jax.ShapeDtypeStruct((N, H), jnp.float32)],
    )(gr2, gc2, he2, hn, ag, cnt, w1a, w1b, w1c, b1, w2, b2,
      nw1a, nw1b, nb1, nw2, nb2)


def _decoder_body(he2, wd1, bd1, wd2t, bd2, out):
    def head(x):
        hid = jnp.maximum(
            jnp.dot(x, wd1[...], preferred_element_type=jnp.float32) + bd1[...],
            0.0)
        logit = jnp.sum(hid * wd2t[...], axis=1, keepdims=True) + bd2[...]
        return 1.0 / (1.0 + jnp.exp(-logit))

    out[...] = 0.5 * (head(he2[:, 0:H]) + head(he2[:, H:2 * H]))


def _tc_decoder(he2, wd1, bd1, wd2t, bd2, blk=2048):
    return pl.pallas_call(
        _decoder_body,
        grid=(E // 2 // blk,),
        in_specs=[
            _rows(blk, 2 * H),
            _full((H, H // 2)), _full((1, H // 2)), _full((1, H // 2)),
            _full((1, 1)),
        ],
        out_specs=_rows(blk, 1),
        out_shape=jax.ShapeDtypeStruct((E // 2, 1), jnp.float32),
    )(he2, wd1, bd1, wd2t, bd2)


def kernel(x, edge_index, batch, damage_locs, W_ne, b_ne, W_ee1, b_ee1,
           W_ee2, b_ee2, W_em1, b_em1, W_em2, b_em2, W_nm1, b_nm1, W_nm2,
           b_nm2, W_d1, b_d1, W_d2, b_d2):
    row = edge_index[0]
    col = edge_index[1]
    row_e, row_o = row[0::2], row[1::2]
    col_e, col_o = col[0::2], col[1::2]
    sc = _sc_kernels()

    dmg_pad = jnp.pad(damage_locs, ((0, 0), (0, 14)))
    dmg_node, = sc["gdmg"](dmg_pad, batch)
    xd = jnp.concatenate(
        [x, dmg_node[:, :2], jnp.zeros((N, 12), jnp.float32)], axis=1)

    xdp = sc["gxd"](xd, row_e, row_o, col_e, col_o)
    h_e2, h_n = _tc_init(xdp, xd, W_ee1, b_ee1.reshape(1, H), W_ee2,
                         b_ee2.reshape(1, H), W_ne, b_ne.reshape(1, H))
    cnt = sc["cnt"](col)

    for l in range(L - 1):
        gr2, gc2, aggr = sc["layer"](h_n, row_e, row_o, col_e, col_o)
        h_e2, h_n = _tc_layer(
            gr2, gc2, h_e2, h_n, aggr, cnt,
            W_em1[l, 0:H], W_em1[l, H:2 * H], W_em1[l, 2 * H:3 * H],
            b_em1[l].reshape(1, H), W_em2[l], b_em2[l].reshape(1, H),
            W_nm1[l, 0:H], W_nm1[l, H:2 * H], b_nm1[l].reshape(1, H),
            W_nm2[l], b_nm2[l].reshape(1, H))

    gr2, gc2 = sc["gpair"](h_n, row_e, row_o, col_e, col_o)
    lf = L - 1
    pred2 = _tc_final(
        gr2, gc2, h_e2,
        W_em1[lf, 0:H], W_em1[lf, H:2 * H], W_em1[lf, 2 * H:3 * H],
        b_em1[lf].reshape(1, H), W_em2[lf], b_em2[lf].reshape(1, H),
        W_d1, b_d1.reshape(1, H // 2), W_d2.reshape(1, H // 2),
        b_d2.reshape(1, 1))
    return pred2.reshape(NB, PAIRS)


# speedup vs baseline: 1.0461x; 1.0461x over previous
import functools

import jax
import jax.numpy as jnp
from jax import lax
from jax.experimental import pallas as pl
from jax.experimental.pallas import tpu as pltpu
from jax.experimental.pallas import tpu_sc as plsc

NB = 4096
PAIRS = 36
NPG = 9
N = NB * NPG
E = NB * PAIRS * 2
H = 64
L = 4

NC = 2
NS = 16
NW = NC * NS
NHALF = N // NC
NPAD = 18944


def _make_gather(n_idx, total, d, ch):
    pw = total // NW
    steps = pw // ch
    assert pw % ch == 0 and ch % 8 == 0

    def body(*refs):
        table = refs[0]
        idxs = refs[1:1 + n_idx]
        outs = refs[1 + n_idx:1 + 2 * n_idx]
        idx_v, rows_v, sem = refs[1 + 2 * n_idx:]
        wid = lax.axis_index("s") * NC + lax.axis_index("c")
        base = wid * pw

        def step(i, carry):
            off = base + i * ch
            for k in range(n_idx):
                pltpu.sync_copy(idxs[k].at[pl.ds(off, ch)], idx_v)
                pltpu.async_copy(table.at[idx_v], rows_v, sem).wait()
                pltpu.sync_copy(rows_v, outs[k].at[pl.ds(off, ch)])
            return carry

        lax.fori_loop(0, steps, step, 0)

    out_type = [jax.ShapeDtypeStruct((total, d), jnp.float32)] * n_idx
    return pl.kernel(
        body,
        out_type=out_type,
        mesh=_mesh(),
        scratch_types=[
            pltpu.VMEM((ch,), jnp.int32),
            pltpu.VMEM((ch, d), jnp.float32),
            pltpu.SemaphoreType.DMA,
        ],
        compiler_params=pltpu.CompilerParams(use_tc_tiling_on_sc=False),
    )


def _fill_rows(ref, rows, d, value):
    def step(i, carry):
        for j in range(d // 16):
            ref[i, pl.ds(j * 16, 16)] = jnp.full((16,), value, jnp.float32)
        return carry
    lax.fori_loop(0, rows, step, 0)


def _make_scatter_add(d, read_rows, ch):
    es = E // NS
    steps = es // ch
    assert es % ch == 0 and ch % 16 == 0
    zr = 32
    per_sub_pad = NPAD // NS
    per_sub_out = NHALF // NS

    def body(*refs):
        if read_rows:
            colidx, grow, out, idx_v, rows_v, zbuf, acc, sem = refs
        else:
            colidx, out, idx_v, rows_v, zbuf, acc, sem = refs
        c = lax.axis_index("c")
        s = lax.axis_index("s")
        nbase = c * NHALF

        _fill_rows(zbuf, zr, d, 0.0)
        if not read_rows:
            _fill_rows(rows_v, ch, d, 1.0)
        for t in range(per_sub_pad // zr):
            pltpu.sync_copy(zbuf, acc.at[pl.ds(s * per_sub_pad + t * zr, zr)])
        plsc.subcore_barrier()

        def step(i, carry):
            off = s * es + i * ch
            pltpu.sync_copy(colidx.at[pl.ds(off, ch)], idx_v)
            if read_rows:
                pltpu.sync_copy(grow.at[pl.ds(off, ch)], rows_v)

            def remap(j, carry2):
                v = idx_v[pl.ds(j * 16, 16)] - nbase
                ok = (v >= 0) & (v < NHALF)
                idx_v[pl.ds(j * 16, 16)] = jnp.where(ok, v, NHALF)
                return carry2

            lax.fori_loop(0, ch // 16, remap, 0)
            pltpu.sync_copy(rows_v, acc.at[idx_v], add=True)
            return carry

        lax.fori_loop(0, steps, step, 0)
        plsc.subcore_barrier()
        pltpu.sync_copy(
            acc.at[pl.ds(s * per_sub_out, per_sub_out)],
            out.at[pl.ds(nbase + s * per_sub_out, per_sub_out)],
        )

    return pl.kernel(
        body,
        out_type=jax.ShapeDtypeStruct((N, d), jnp.float32),
        mesh=_mesh(),
        scratch_types=[
            pltpu.VMEM((ch,), jnp.int32),
            pltpu.VMEM((ch, d), jnp.float32),
            pltpu.VMEM((zr, d), jnp.float32),
            pltpu.VMEM_SHARED((NPAD, d), jnp.float32),
            pltpu.SemaphoreType.DMA,
        ],
        compiler_params=pltpu.CompilerParams(use_tc_tiling_on_sc=False),
    )


def _make_layer_sc(chp):
    p2 = (E // 2) // NC
    po = p2 // NS
    steps = po // chp
    assert po % chp == 0 and chp % 16 == 0
    zr = 32
    per_sub_pad = NPAD // NS
    per_sub_out = NHALF // NS

    def body(hn, row_e, row_o, col_e, col_o, gr2, gc2, out_aggr,
             i_re, i_ro, i_ce, i_co, rows_re, rows_ro, rows_ce, rows_co,
             zbuf, acc, sem1, sem2, sem3, sem4, sem_w):
        c = lax.axis_index("c")
        s = lax.axis_index("s")
        nbase = c * NHALF

        _fill_rows(zbuf, zr, H, 0.0)
        for t in range(per_sub_pad // zr):
            pltpu.sync_copy(zbuf, acc.at[pl.ds(s * per_sub_pad + t * zr, zr)])
        plsc.subcore_barrier()

        own_base = c * p2 + s * po
        for_base = (p2 - c * p2) + s * po

        def remap(idx_ref):
            def rstep(j, carry):
                v = idx_ref[pl.ds(j * 16, 16)] - nbase
                ok = (v >= 0) & (v < NHALF)
                idx_ref[pl.ds(j * 16, 16)] = jnp.where(ok, v, NHALF)
                return carry
            lax.fori_loop(0, chp // 16, rstep, 0)

        def own_step(i, carry):
            off = own_base + i * chp
            pltpu.sync_copy(row_e.at[pl.ds(off, chp)], i_re)
            g1 = pltpu.async_copy(hn.at[i_re], rows_re, sem1)
            pltpu.sync_copy(row_o.at[pl.ds(off, chp)], i_ro)
            g2 = pltpu.async_copy(hn.at[i_ro], rows_ro, sem2)
            pltpu.sync_copy(col_e.at[pl.ds(off, chp)], i_ce)
            g3 = pltpu.async_copy(hn.at[i_ce], rows_ce, sem3)
            pltpu.sync_copy(col_o.at[pl.ds(off, chp)], i_co)
            g4 = pltpu.async_copy(hn.at[i_co], rows_co, sem4)
            g1.wait()
            w1 = pltpu.async_copy(
                rows_re, gr2.at[pl.ds(off, chp), pl.ds(0, H)], sem_w)
            g2.wait()
            w2 = pltpu.async_copy(
                rows_ro, gr2.at[pl.ds(off, chp), pl.ds(H, H)], sem_w)
            g3.wait()
            w3 = pltpu.async_copy(
                rows_ce, gc2.at[pl.ds(off, chp), pl.ds(0, H)], sem_w)
            g4.wait()
            w4 = pltpu.async_copy(
                rows_co, gc2.at[pl.ds(off, chp), pl.ds(H, H)], sem_w)
            remap(i_ce)
            remap(i_co)
            pltpu.sync_copy(rows_re, acc.at[i_ce], add=True)
            pltpu.sync_copy(rows_ro, acc.at[i_co], add=True)
            w1.wait()
            w2.wait()
            w3.wait()
            w4.wait()
            return carry

        def foreign_step(i, carry):
            off = for_base + i * chp
            pltpu.sync_copy(row_e.at[pl.ds(off, chp)], i_re)
            g1 = pltpu.async_copy(hn.at[i_re], rows_re, sem1)
            pltpu.sync_copy(row_o.at[pl.ds(off, chp)], i_ro)
            g2 = pltpu.async_copy(hn.at[i_ro], rows_ro, sem2)
            pltpu.sync_copy(col_e.at[pl.ds(off, chp)], i_ce)
            pltpu.sync_copy(col_o.at[pl.ds(off, chp)], i_co)
            remap(i_ce)
            remap(i_co)
            g1.wait()
            pltpu.sync_copy(rows_re, acc.at[i_ce], add=True)
            g2.wait()
            pltpu.sync_copy(rows_ro, acc.at[i_co], add=True)
            return carry

        lax.fori_loop(0, steps, own_step, 0)
        lax.fori_loop(0, steps, foreign_step, 0)
        plsc.subcore_barrier()
        pltpu.sync_copy(
            acc.at[pl.ds(s * per_sub_out, per_sub_out)],
            out_aggr.at[pl.ds(nbase + s * per_sub_out, per_sub_out)],
        )

    return pl.kernel(
        body,
        out_type=[
            jax.ShapeDtypeStruct((E // 2, 2 * H), jnp.float32),
            jax.ShapeDtypeStruct((E // 2, 2 * H), jnp.float32),
            jax.ShapeDtypeStruct((N, H), jnp.float32),
        ],
        mesh=_mesh(),
        scratch_types=[
            pltpu.VMEM((chp,), jnp.int32),
            pltpu.VMEM((chp,), jnp.int32),
            pltpu.VMEM((chp,), jnp.int32),
            pltpu.VMEM((chp,), jnp.int32),
            pltpu.VMEM((chp, H), jnp.float32),
            pltpu.VMEM((chp, H), jnp.float32),
            pltpu.VMEM((chp, H), jnp.float32),
            pltpu.VMEM((chp, H), jnp.float32),
            pltpu.VMEM((zr, H), jnp.float32),
            pltpu.VMEM_SHARED((NPAD, H), jnp.float32),
            pltpu.SemaphoreType.DMA,
            pltpu.SemaphoreType.DMA,
            pltpu.SemaphoreType.DMA,
            pltpu.SemaphoreType.DMA,
            pltpu.SemaphoreType.DMA,
        ],
        compiler_params=pltpu.CompilerParams(use_tc_tiling_on_sc=False),
    )


def _make_gxd_packed(chp):
    pw = (E // 2) // NW
    steps = pw // chp
    assert pw % chp == 0 and chp % 8 == 0
    d = 16

    def body(xd, row_e, row_o, col_e, col_o, out,
             i_re, i_ro, i_ce, i_co, r1, r2, r3, r4,
             sem1, sem2, sem3, sem4, sem_w):
        wid = lax.axis_index("s") * NC + lax.axis_index("c")
        base = wid * pw

        def step(i, carry):
            off = base + i * chp
            pltpu.sync_copy(row_e.at[pl.ds(off, chp)], i_re)
            g1 = pltpu.async_copy(xd.at[i_re], r1, sem1)
            pltpu.sync_copy(row_o.at[pl.ds(off, chp)], i_ro)
            g2 = pltpu.async_copy(xd.at[i_ro], r2, sem2)
            pltpu.sync_copy(col_e.at[pl.ds(off, chp)], i_ce)
            g3 = pltpu.async_copy(xd.at[i_ce], r3, sem3)
            pltpu.sync_copy(col_o.at[pl.ds(off, chp)], i_co)
            g4 = pltpu.async_copy(xd.at[i_co], r4, sem4)
            g1.wait()
            w1 = pltpu.async_copy(
                r1, out.at[pl.ds(off, chp), pl.ds(0, d)], sem_w)
            g2.wait()
            w2 = pltpu.async_copy(
                r2, out.at[pl.ds(off, chp), pl.ds(d, d)], sem_w)
            g3.wait()
            w3 = pltpu.async_copy(
                r3, out.at[pl.ds(off, chp), pl.ds(2 * d, d)], sem_w)
            g4.wait()
            w4 = pltpu.async_copy(
                r4, out.at[pl.ds(off, chp), pl.ds(3 * d, d)], sem_w)
            w1.wait()
            w2.wait()
            w3.wait()
            w4.wait()
            return carry

        lax.fori_loop(0, steps, step, 0)

    return pl.kernel(
        body,
        out_type=jax.ShapeDtypeStruct((E // 2, 2 * H), jnp.float32),
        mesh=_mesh(),
        scratch_types=(
            [pltpu.VMEM((chp,), jnp.int32)] * 4
            + [pltpu.VMEM((chp, d), jnp.float32)] * 4
            + [pltpu.SemaphoreType.DMA] * 5
        ),
        compiler_params=pltpu.CompilerParams(use_tc_tiling_on_sc=False),
    )


def _make_gpair(chp):
    pw = (E // 2) // NW
    steps = pw // chp
    assert pw % chp == 0 and chp % 8 == 0

    def body(hn, row_e, row_o, col_e, col_o, gr2, gc2,
             i_re, i_ro, i_ce, i_co, rows_re, rows_ro, rows_ce, rows_co,
             sem1, sem2, sem3, sem4, sem_w):
        wid = lax.axis_index("s") * NC + lax.axis_index("c")
        base = wid * pw

        def step(i, carry):
            off = base + i * chp
            pltpu.sync_copy(row_e.at[pl.ds(off, chp)], i_re)
            g1 = pltpu.async_copy(hn.at[i_re], rows_re, sem1)
            pltpu.sync_copy(row_o.at[pl.ds(off, chp)], i_ro)
            g2 = pltpu.async_copy(hn.at[i_ro], rows_ro, sem2)
            pltpu.sync_copy(col_e.at[pl.ds(off, chp)], i_ce)
            g3 = pltpu.async_copy(hn.at[i_ce], rows_ce, sem3)
            pltpu.sync_copy(col_o.at[pl.ds(off, chp)], i_co)
            g4 = pltpu.async_copy(hn.at[i_co], rows_co, sem4)
            g1.wait()
            w1 = pltpu.async_copy(
                rows_re, gr2.at[pl.ds(off, chp), pl.ds(0, H)], sem_w)
            g2.wait()
            w2 = pltpu.async_copy(
                rows_ro, gr2.at[pl.ds(off, chp), pl.ds(H, H)], sem_w)
            g3.wait()
            w3 = pltpu.async_copy(
                rows_ce, gc2.at[pl.ds(off, chp), pl.ds(0, H)], sem_w)
            g4.wait()
            w4 = pltpu.async_copy(
                rows_co, gc2.at[pl.ds(off, chp), pl.ds(H, H)], sem_w)
            w1.wait()
            w2.wait()
            w3.wait()
            w4.wait()
            return carry

        lax.fori_loop(0, steps, step, 0)

    return pl.kernel(
        body,
        out_type=[
            jax.ShapeDtypeStruct((E // 2, 2 * H), jnp.float32),
            jax.ShapeDtypeStruct((E // 2, 2 * H), jnp.float32),
        ],
        mesh=_mesh(),
        scratch_types=[
            pltpu.VMEM((chp,), jnp.int32),
            pltpu.VMEM((chp,), jnp.int32),
            pltpu.VMEM((chp,), jnp.int32),
            pltpu.VMEM((chp,), jnp.int32),
            pltpu.VMEM((chp, H), jnp.float32),
            pltpu.VMEM((chp, H), jnp.float32),
            pltpu.VMEM((chp, H), jnp.float32),
            pltpu.VMEM((chp, H), jnp.float32),
            pltpu.SemaphoreType.DMA,
            pltpu.SemaphoreType.DMA,
            pltpu.SemaphoreType.DMA,
            pltpu.SemaphoreType.DMA,
            pltpu.SemaphoreType.DMA,
        ],
        compiler_params=pltpu.CompilerParams(use_tc_tiling_on_sc=False),
    )


@functools.cache
def _mesh():
    return plsc.VectorSubcoreMesh(core_axis_name="c", subcore_axis_name="s",
                                  num_cores=NC, num_subcores=NS)


@functools.cache
def _sc_kernels():
    return {
        "gdmg": _make_gather(1, N, 16, 1152),
        "gxd": _make_gxd_packed(512),
        "cnt": _make_scatter_add(16, False, 512),
        "layer": _make_layer_sc(192),
        "gpair": _make_gpair(288),
    }


BEP = 1024
BN = 2048
GE_ = (E // 2) // BEP
GN_ = N // BN


def _full(shape):
    return pl.BlockSpec(shape, lambda i: tuple(0 for _ in shape))


def _rows(blk, d):
    return pl.BlockSpec((blk, d), lambda i: (i, 0))


def _espec(d):
    return pl.BlockSpec((BEP, d), lambda i: (jnp.minimum(i, GE_ - 1), 0))


def _nspec(d):
    return pl.BlockSpec(
        (BN, d), lambda i: (jnp.clip(i - GE_, 0, GN_ - 1), 0))


def _edge_init_math(xdr, xdc, w1, b1, w2, b2):
    eps = 1e-8
    a = jnp.transpose(xdr)
    c = jnp.transpose(xdc)
    sx0, sx1, dg0, dg1 = a[0:1, :], a[1:2, :], a[2:3, :], a[3:4, :]
    dx0, dx1 = c[0:1, :], c[1:2, :]
    v0 = sx0 - dx0
    v1 = sx1 - dx1
    l2r = v0 * v0 + v1 * v1
    el = jnp.sqrt(l2r + eps)
    l2 = jnp.maximum(l2r, eps)
    t = jnp.clip(((dg0 - sx0) * (dx0 - sx0) + (dg1 - sx1) * (dx1 - sx1)) / l2,
                 0.0, 1.0)
    p0 = sx0 + t * (dx0 - sx0)
    p1 = sx1 + t * (dx1 - sx1)
    dfd = jnp.sqrt((dg0 - p0) ** 2 + (dg1 - p1) ** 2 + eps)
    dtx = jnp.sqrt((sx0 - dg0) ** 2 + (sx1 - dg1) ** 2 + eps)
    drx = jnp.sqrt((dx0 - dg0) ** 2 + (dx1 - dg1) ** 2 + eps)
    phys_t = jnp.concatenate([v0, v1, el, dfd, dtx, drx], axis=0)
    pre = lax.dot_general(phys_t, w1[...], (((0,), (0,)), ((), ())),
                          preferred_element_type=jnp.float32) + b1[...]
    hid = jnp.maximum(pre, 0.0)
    return jnp.dot(hid, w2[...], preferred_element_type=jnp.float32) + b2[...]


def _init_body(xdp, xd, w1, b1, w2, b2, wn, bn, out_he2, out_hn):
    pid = pl.program_id(0)

    @pl.when(pid < GE_)
    def _():
        he_e = _edge_init_math(xdp[:, 0:16], xdp[:, 32:48], w1, b1, w2, b2)
        he_o = _edge_init_math(xdp[:, 16:32], xdp[:, 48:64], w1, b1, w2, b2)
        out_he2[...] = jnp.concatenate([he_e, he_o], axis=1)

    @pl.when(pid >= GE_)
    def _():
        out_hn[...] = (xd[:, 0:1] * wn[0:1, :] + xd[:, 1:2] * wn[1:2, :]
                       + bn[...])


def _tc_init(xdp, xd, w1, b1, w2, b2, wn, bn):
    return pl.pallas_call(
        _init_body,
        grid=(GE_ + GN_,),
        in_specs=[
            _espec(2 * H), _nspec(16),
            _full((6, H)), _full((1, H)), _full((H, H)), _full((1, H)),
            _full((2, H)), _full((1, H)),
        ],
        out_specs=[_espec(2 * H), _nspec(H)],
        out_shape=[jax.ShapeDtypeStruct((E // 2, 2 * H), jnp.float32),
                   jax.ShapeDtypeStruct((N, H), jnp.float32)],
    )(xdp, xd, w1, b1, w2, b2, wn, bn)


def _layer_body(gr2, gc2, he2, hn, ag, cnt, w1a, w1b, w1c, b1, w2, b2,
                nw1a, nw1b, nb1, nw2, nb2, out_he2, out_hn):
    pid = pl.program_id(0)

    @pl.when(pid < GE_)
    def _():
        def upd(g_r, g_c, h):
            pre = (jnp.dot(g_r, w1a[...], preferred_element_type=jnp.float32)
                   + jnp.dot(g_c, w1b[...], preferred_element_type=jnp.float32)
                   + jnp.dot(h, w1c[...], preferred_element_type=jnp.float32)
                   + b1[...])
            hid = jnp.maximum(pre, 0.0)
            return h + jnp.dot(hid, w2[...],
                               preferred_element_type=jnp.float32) + b2[...]

        even = upd(gr2[:, 0:H], gc2[:, 0:H], he2[:, 0:H])
        odd = upd(gr2[:, H:2 * H], gc2[:, H:2 * H], he2[:, H:2 * H])
        out_he2[...] = jnp.concatenate([even, odd], axis=1)

    @pl.when(pid >= GE_)
    def _():
        aggr = ag[...] / jnp.maximum(cnt[:, 0:1], 1.0)
        pre = (jnp.dot(hn[...], nw1a[...], preferred_element_type=jnp.float32)
               + jnp.dot(aggr, nw1b[...], preferred_element_type=jnp.float32)
               + nb1[...])
        hid = jnp.maximum(pre, 0.0)
        out_hn[...] = hn[...] + jnp.dot(
            hid, nw2[...], preferred_element_type=jnp.float32) + nb2[...]


def _final_body(gr2, gc2, he2, w1a, w1b, w1c, b1, w2, b2,
                wd1, bd1, wd2t, bd2, out_pred):
    def upd(g_r, g_c, h):
        pre = (jnp.dot(g_r, w1a[...], preferred_element_type=jnp.float32)
               + jnp.dot(g_c, w1b[...], preferred_element_type=jnp.float32)
               + jnp.dot(h, w1c[...], preferred_element_type=jnp.float32)
               + b1[...])
        hid = jnp.maximum(pre, 0.0)
        return h + jnp.dot(hid, w2[...],
                           preferred_element_type=jnp.float32) + b2[...]

    def head(x):
        hid = jnp.maximum(
            jnp.dot(x, wd1[...], preferred_element_type=jnp.float32)
            + bd1[...], 0.0)
        logit = jnp.sum(hid * wd2t[...], axis=1, keepdims=True) + bd2[...]
        return 1.0 / (1.0 + jnp.exp(-logit))

    even = upd(gr2[:, 0:H], gc2[:, 0:H], he2[:, 0:H])
    odd = upd(gr2[:, H:2 * H], gc2[:, H:2 * H], he2[:, H:2 * H])
    out_pred[...] = 0.5 * (head(even) + head(odd))


def _tc_final(gr2, gc2, he2, w1a, w1b, w1c, b1, w2, b2, wd1, bd1, wd2t, bd2):
    return pl.pallas_call(
        _final_body,
        grid=(GE_,),
        in_specs=[
            _rows(BEP, 2 * H), _rows(BEP, 2 * H), _rows(BEP, 2 * H),
            _full((H, H)), _full((H, H)), _full((H, H)), _full((1, H)),
            _full((H, H)), _full((1, H)),
            _full((H, H // 2)), _full((1, H // 2)), _full((1, H // 2)),
            _full((1, 1)),
        ],
        out_specs=_rows(BEP, 1),
        out_shape=jax.ShapeDtypeStruct((E // 2, 1), jnp.float32),
    )(gr2, gc2, he2, w1a, w1b, w1c, b1, w2, b2, wd1, bd1, wd2t, bd2)


def _tc_layer(gr2, gc2, he2, hn, ag, cnt, w1a, w1b, w1c, b1, w2, b2,
              nw1a, nw1b, nb1, nw2, nb2):
    return pl.pallas_call(
        _layer_body,
        grid=(GE_ + GN_,),
        in_specs=[
            _espec(2 * H), _espec(2 * H), _espec(2 * H),
            _nspec(H), _nspec(H), _nspec(16),
            _full((H, H)), _full((H, H)), _full((H, H)), _full((1, H)),
            _full((H, H)), _full((1, H)),
            _full((H, H)), _full((H, H)), _full((1, H)),
            _full((H, H)), _full((1, H)),
        ],
        out_specs=[_espec(2 * H), _nspec(H)],
        out_shape=[jax.ShapeDtypeStruct((E // 2, 2 * H), jnp.float32),
                   jax.ShapeDtypeStruct((N, H), jnp.float32)],
    )(gr2, gc2, he2, hn, ag, cnt, w1a, w1b, w1c, b1, w2, b2,
      nw1a, nw1b, nb1, nw2, nb2)


def _decoder_body(he2, wd1, bd1, wd2t, bd2, out):
    def head(x):
        hid = jnp.maximum(
            jnp.dot(x, wd1[...], preferred_element_type=jnp.float32) + bd1[...],
            0.0)
        logit = jnp.sum(hid * wd2t[...], axis=1, keepdims=True) + bd2[...]
        return 1.0 / (1.0 + jnp.exp(-logit))

    out[...] = 0.5 * (head(he2[:, 0:H]) + head(he2[:, H:2 * H]))


def _tc_decoder(he2, wd1, bd1, wd2t, bd2, blk=2048):
    return pl.pallas_call(
        _decoder_body,
        grid=(E // 2 // blk,),
        in_specs=[
            _rows(blk, 2 * H),
            _full((H, H // 2)), _full((1, H // 2)), _full((1, H // 2)),
            _full((1, 1)),
        ],
        out_specs=_rows(blk, 1),
        out_shape=jax.ShapeDtypeStruct((E // 2, 1), jnp.float32),
    )(he2, wd1, bd1, wd2t, bd2)


def kernel(x, edge_index, batch, damage_locs, W_ne, b_ne, W_ee1, b_ee1,
           W_ee2, b_ee2, W_em1, b_em1, W_em2, b_em2, W_nm1, b_nm1, W_nm2,
           b_nm2, W_d1, b_d1, W_d2, b_d2):
    row = edge_index[0]
    col = edge_index[1]
    row_e, row_o = row[0::2], row[1::2]
    col_e, col_o = col[0::2], col[1::2]
    sc = _sc_kernels()

    dmg_pad = jnp.pad(damage_locs, ((0, 0), (0, 14)))
    dmg_node, = sc["gdmg"](dmg_pad, batch)
    xd = jnp.concatenate(
        [x, dmg_node[:, :2], jnp.zeros((N, 12), jnp.float32)], axis=1)

    xdp = sc["gxd"](xd, row_e, row_o, col_e, col_o)
    h_e2, h_n = _tc_init(xdp, xd, W_ee1, b_ee1.reshape(1, H), W_ee2,
                         b_ee2.reshape(1, H), W_ne, b_ne.reshape(1, H))
    cnt = sc["cnt"](col)

    for l in range(L - 1):
        gr2, gc2, aggr = sc["layer"](h_n, row_e, row_o, col_e, col_o)
        h_e2, h_n = _tc_layer(
            gr2, gc2, h_e2, h_n, aggr, cnt,
            W_em1[l, 0:H], W_em1[l, H:2 * H], W_em1[l, 2 * H:3 * H],
            b_em1[l].reshape(1, H), W_em2[l], b_em2[l].reshape(1, H),
            W_nm1[l, 0:H], W_nm1[l, H:2 * H], b_nm1[l].reshape(1, H),
            W_nm2[l], b_nm2[l].reshape(1, H))

    gr2, gc2 = sc["gpair"](h_n, row_e, row_o, col_e, col_o)
    lf = L - 1
    pred2 = _tc_final(
        gr2, gc2, h_e2,
        W_em1[lf, 0:H], W_em1[lf, H:2 * H], W_em1[lf, 2 * H:3 * H],
        b_em1[lf].reshape(1, H), W_em2[lf], b_em2[lf].reshape(1, H),
        W_d1, b_d1.reshape(1, H // 2), W_d2.reshape(1, H // 2),
        b_d2.reshape(1, 1))
    return pred2.reshape(NB, PAIRS)


# speedup vs baseline: 1.0492x; 1.0030x over previous
import functools

import jax
import jax.numpy as jnp
from jax import lax
from jax.experimental import pallas as pl
from jax.experimental.pallas import tpu as pltpu
from jax.experimental.pallas import tpu_sc as plsc

NB = 4096
PAIRS = 36
NPG = 9
N = NB * NPG
E = NB * PAIRS * 2
H = 64
L = 4

NC = 2
NS = 16
NW = NC * NS
NHALF = N // NC
NPAD = 18944


def _make_gather(n_idx, total, d, ch):
    pw = total // NW
    steps = pw // ch
    assert pw % ch == 0 and ch % 8 == 0

    def body(*refs):
        table = refs[0]
        idxs = refs[1:1 + n_idx]
        outs = refs[1 + n_idx:1 + 2 * n_idx]
        idx_v, rows_v, sem = refs[1 + 2 * n_idx:]
        wid = lax.axis_index("s") * NC + lax.axis_index("c")
        base = wid * pw

        def step(i, carry):
            off = base + i * ch
            for k in range(n_idx):
                pltpu.sync_copy(idxs[k].at[pl.ds(off, ch)], idx_v)
                pltpu.async_copy(table.at[idx_v], rows_v, sem).wait()
                pltpu.sync_copy(rows_v, outs[k].at[pl.ds(off, ch)])
            return carry

        lax.fori_loop(0, steps, step, 0)

    out_type = [jax.ShapeDtypeStruct((total, d), jnp.float32)] * n_idx
    return pl.kernel(
        body,
        out_type=out_type,
        mesh=_mesh(),
        scratch_types=[
            pltpu.VMEM((ch,), jnp.int32),
            pltpu.VMEM((ch, d), jnp.float32),
            pltpu.SemaphoreType.DMA,
        ],
        compiler_params=pltpu.CompilerParams(use_tc_tiling_on_sc=False),
    )


def _fill_rows(ref, rows, d, value):
    def step(i, carry):
        for j in range(d // 16):
            ref[i, pl.ds(j * 16, 16)] = jnp.full((16,), value, jnp.float32)
        return carry
    lax.fori_loop(0, rows, step, 0)


def _make_scatter_add(d, read_rows, ch):
    es = E // NS
    steps = es // ch
    assert es % ch == 0 and ch % 16 == 0
    zr = 32
    per_sub_pad = NPAD // NS
    per_sub_out = NHALF // NS

    def body(*refs):
        if read_rows:
            colidx, grow, out, idx_v, rows_v, zbuf, acc, sem = refs
        else:
            colidx, out, idx_v, rows_v, zbuf, acc, sem = refs
        c = lax.axis_index("c")
        s = lax.axis_index("s")
        nbase = c * NHALF

        _fill_rows(zbuf, zr, d, 0.0)
        if not read_rows:
            _fill_rows(rows_v, ch, d, 1.0)
        for t in range(per_sub_pad // zr):
            pltpu.sync_copy(zbuf, acc.at[pl.ds(s * per_sub_pad + t * zr, zr)])
        plsc.subcore_barrier()

        def step(i, carry):
            off = s * es + i * ch
            pltpu.sync_copy(colidx.at[pl.ds(off, ch)], idx_v)
            if read_rows:
                pltpu.sync_copy(grow.at[pl.ds(off, ch)], rows_v)

            def remap(j, carry2):
                v = idx_v[pl.ds(j * 16, 16)] - nbase
                ok = (v >= 0) & (v < NHALF)
                idx_v[pl.ds(j * 16, 16)] = jnp.where(ok, v, NHALF)
                return carry2

            lax.fori_loop(0, ch // 16, remap, 0)
            pltpu.sync_copy(rows_v, acc.at[idx_v], add=True)
            return carry

        lax.fori_loop(0, steps, step, 0)
        plsc.subcore_barrier()
        pltpu.sync_copy(
            acc.at[pl.ds(s * per_sub_out, per_sub_out)],
            out.at[pl.ds(nbase + s * per_sub_out, per_sub_out)],
        )

    return pl.kernel(
        body,
        out_type=jax.ShapeDtypeStruct((N, d), jnp.float32),
        mesh=_mesh(),
        scratch_types=[
            pltpu.VMEM((ch,), jnp.int32),
            pltpu.VMEM((ch, d), jnp.float32),
            pltpu.VMEM((zr, d), jnp.float32),
            pltpu.VMEM_SHARED((NPAD, d), jnp.float32),
            pltpu.SemaphoreType.DMA,
        ],
        compiler_params=pltpu.CompilerParams(use_tc_tiling_on_sc=False),
    )


def _make_layer_sc(chp):
    p2 = (E // 2) // NC
    po = p2 // NS
    steps = po // chp
    assert po % chp == 0 and chp % 16 == 0
    zr = 32
    per_sub_pad = NPAD // NS
    per_sub_out = NHALF // NS

    def body(hn, row_e, row_o, col_e, col_o, gr2, gc2, out_aggr,
             i_re, i_ro, i_ce, i_co, rows_re, rows_ro, rows_ce, rows_co,
             zbuf, acc, sem1, sem2, sem3, sem4, sem_w):
        c = lax.axis_index("c")
        s = lax.axis_index("s")
        nbase = c * NHALF

        _fill_rows(zbuf, zr, H, 0.0)
        for t in range(per_sub_pad // zr):
            pltpu.sync_copy(zbuf, acc.at[pl.ds(s * per_sub_pad + t * zr, zr)])
        plsc.subcore_barrier()

        own_base = c * p2 + s * po
        for_base = (p2 - c * p2) + s * po

        def remap(idx_ref):
            def rstep(j, carry):
                v = idx_ref[pl.ds(j * 16, 16)] - nbase
                ok = (v >= 0) & (v < NHALF)
                idx_ref[pl.ds(j * 16, 16)] = jnp.where(ok, v, NHALF)
                return carry
            lax.fori_loop(0, chp // 16, rstep, 0)

        def own_step(i, carry):
            off = own_base + i * chp
            pltpu.sync_copy(row_e.at[pl.ds(off, chp)], i_re)
            g1 = pltpu.async_copy(hn.at[i_re], rows_re, sem1)
            pltpu.sync_copy(row_o.at[pl.ds(off, chp)], i_ro)
            g2 = pltpu.async_copy(hn.at[i_ro], rows_ro, sem2)
            pltpu.sync_copy(col_e.at[pl.ds(off, chp)], i_ce)
            g3 = pltpu.async_copy(hn.at[i_ce], rows_ce, sem3)
            pltpu.sync_copy(col_o.at[pl.ds(off, chp)], i_co)
            g4 = pltpu.async_copy(hn.at[i_co], rows_co, sem4)
            g1.wait()
            w1 = pltpu.async_copy(
                rows_re, gr2.at[pl.ds(off, chp), pl.ds(0, H)], sem_w)
            g2.wait()
            w2 = pltpu.async_copy(
                rows_ro, gr2.at[pl.ds(off, chp), pl.ds(H, H)], sem_w)
            g3.wait()
            w3 = pltpu.async_copy(
                rows_ce, gc2.at[pl.ds(off, chp), pl.ds(0, H)], sem_w)
            g4.wait()
            w4 = pltpu.async_copy(
                rows_co, gc2.at[pl.ds(off, chp), pl.ds(H, H)], sem_w)
            remap(i_ce)
            remap(i_co)
            pltpu.sync_copy(rows_re, acc.at[i_ce], add=True)
            pltpu.sync_copy(rows_ro, acc.at[i_co], add=True)
            w1.wait()
            w2.wait()
            w3.wait()
            w4.wait()
            return carry

        def foreign_step(i, carry):
            off = for_base + i * chp
            pltpu.sync_copy(row_e.at[pl.ds(off, chp)], i_re)
            g1 = pltpu.async_copy(hn.at[i_re], rows_re, sem1)
            pltpu.sync_copy(row_o.at[pl.ds(off, chp)], i_ro)
            g2 = pltpu.async_copy(hn.at[i_ro], rows_ro, sem2)
            pltpu.sync_copy(col_e.at[pl.ds(off, chp)], i_ce)
            pltpu.sync_copy(col_o.at[pl.ds(off, chp)], i_co)
            remap(i_ce)
            remap(i_co)
            g1.wait()
            pltpu.sync_copy(rows_re, acc.at[i_ce], add=True)
            g2.wait()
            pltpu.sync_copy(rows_ro, acc.at[i_co], add=True)
            return carry

        lax.fori_loop(0, steps, own_step, 0)
        lax.fori_loop(0, steps, foreign_step, 0)
        plsc.subcore_barrier()
        pltpu.sync_copy(
            acc.at[pl.ds(s * per_sub_out, per_sub_out)],
            out_aggr.at[pl.ds(nbase + s * per_sub_out, per_sub_out)],
        )

    return pl.kernel(
        body,
        out_type=[
            jax.ShapeDtypeStruct((E // 2, 2 * H), jnp.float32),
            jax.ShapeDtypeStruct((E // 2, 2 * H), jnp.float32),
            jax.ShapeDtypeStruct((N, H), jnp.float32),
        ],
        mesh=_mesh(),
        scratch_types=[
            pltpu.VMEM((chp,), jnp.int32),
            pltpu.VMEM((chp,), jnp.int32),
            pltpu.VMEM((chp,), jnp.int32),
            pltpu.VMEM((chp,), jnp.int32),
            pltpu.VMEM((chp, H), jnp.float32),
            pltpu.VMEM((chp, H), jnp.float32),
            pltpu.VMEM((chp, H), jnp.float32),
            pltpu.VMEM((chp, H), jnp.float32),
            pltpu.VMEM((zr, H), jnp.float32),
            pltpu.VMEM_SHARED((NPAD, H), jnp.float32),
            pltpu.SemaphoreType.DMA,
            pltpu.SemaphoreType.DMA,
            pltpu.SemaphoreType.DMA,
            pltpu.SemaphoreType.DMA,
            pltpu.SemaphoreType.DMA,
        ],
        compiler_params=pltpu.CompilerParams(use_tc_tiling_on_sc=False),
    )


def _make_gxd_packed(chp):
    pw = (E // 2) // NW
    steps = pw // chp
    assert pw % chp == 0 and chp % 8 == 0
    d = 16

    def body(xd, row_e, row_o, col_e, col_o, out,
             i_re, i_ro, i_ce, i_co, r1, r2, r3, r4,
             sem1, sem2, sem3, sem4, sem_w):
        wid = lax.axis_index("s") * NC + lax.axis_index("c")
        base = wid * pw

        def step(i, carry):
            off = base + i * chp
            pltpu.sync_copy(row_e.at[pl.ds(off, chp)], i_re)
            g1 = pltpu.async_copy(xd.at[i_re], r1, sem1)
            pltpu.sync_copy(row_o.at[pl.ds(off, chp)], i_ro)
            g2 = pltpu.async_copy(xd.at[i_ro], r2, sem2)
            pltpu.sync_copy(col_e.at[pl.ds(off, chp)], i_ce)
            g3 = pltpu.async_copy(xd.at[i_ce], r3, sem3)
            pltpu.sync_copy(col_o.at[pl.ds(off, chp)], i_co)
            g4 = pltpu.async_copy(xd.at[i_co], r4, sem4)
            g1.wait()
            w1 = pltpu.async_copy(
                r1, out.at[pl.ds(off, chp), pl.ds(0, d)], sem_w)
            g2.wait()
            w2 = pltpu.async_copy(
                r2, out.at[pl.ds(off, chp), pl.ds(d, d)], sem_w)
            g3.wait()
            w3 = pltpu.async_copy(
                r3, out.at[pl.ds(off, chp), pl.ds(2 * d, d)], sem_w)
            g4.wait()
            w4 = pltpu.async_copy(
                r4, out.at[pl.ds(off, chp), pl.ds(3 * d, d)], sem_w)
            w1.wait()
            w2.wait()
            w3.wait()
            w4.wait()
            return carry

        lax.fori_loop(0, steps, step, 0)

    return pl.kernel(
        body,
        out_type=jax.ShapeDtypeStruct((E // 2, 2 * H), jnp.float32),
        mesh=_mesh(),
        scratch_types=(
            [pltpu.VMEM((chp,), jnp.int32)] * 4
            + [pltpu.VMEM((chp, d), jnp.float32)] * 4
            + [pltpu.SemaphoreType.DMA] * 5
        ),
        compiler_params=pltpu.CompilerParams(use_tc_tiling_on_sc=False),
    )


def _make_gpair(chp):
    pw = (E // 2) // NW
    steps = pw // chp
    assert pw % chp == 0 and chp % 8 == 0

    def body(hn, row_e, row_o, col_e, col_o, gr2, gc2,
             i_re, i_ro, i_ce, i_co, rows_re, rows_ro, rows_ce, rows_co,
             sem1, sem2, sem3, sem4, sem_w):
        wid = lax.axis_index("s") * NC + lax.axis_index("c")
        base = wid * pw

        def step(i, carry):
            off = base + i * chp
            pltpu.sync_copy(row_e.at[pl.ds(off, chp)], i_re)
            g1 = pltpu.async_copy(hn.at[i_re], rows_re, sem1)
            pltpu.sync_copy(row_o.at[pl.ds(off, chp)], i_ro)
            g2 = pltpu.async_copy(hn.at[i_ro], rows_ro, sem2)
            pltpu.sync_copy(col_e.at[pl.ds(off, chp)], i_ce)
            g3 = pltpu.async_copy(hn.at[i_ce], rows_ce, sem3)
            pltpu.sync_copy(col_o.at[pl.ds(off, chp)], i_co)
            g4 = pltpu.async_copy(hn.at[i_co], rows_co, sem4)
            g1.wait()
            w1 = pltpu.async_copy(
                rows_re, gr2.at[pl.ds(off, chp), pl.ds(0, H)], sem_w)
            g2.wait()
            w2 = pltpu.async_copy(
                rows_ro, gr2.at[pl.ds(off, chp), pl.ds(H, H)], sem_w)
            g3.wait()
            w3 = pltpu.async_copy(
                rows_ce, gc2.at[pl.ds(off, chp), pl.ds(0, H)], sem_w)
            g4.wait()
            w4 = pltpu.async_copy(
                rows_co, gc2.at[pl.ds(off, chp), pl.ds(H, H)], sem_w)
            w1.wait()
            w2.wait()
            w3.wait()
            w4.wait()
            return carry

        lax.fori_loop(0, steps, step, 0)

    return pl.kernel(
        body,
        out_type=[
            jax.ShapeDtypeStruct((E // 2, 2 * H), jnp.float32),
            jax.ShapeDtypeStruct((E // 2, 2 * H), jnp.float32),
        ],
        mesh=_mesh(),
        scratch_types=[
            pltpu.VMEM((chp,), jnp.int32),
            pltpu.VMEM((chp,), jnp.int32),
            pltpu.VMEM((chp,), jnp.int32),
            pltpu.VMEM((chp,), jnp.int32),
            pltpu.VMEM((chp, H), jnp.float32),
            pltpu.VMEM((chp, H), jnp.float32),
            pltpu.VMEM((chp, H), jnp.float32),
            pltpu.VMEM((chp, H), jnp.float32),
            pltpu.SemaphoreType.DMA,
            pltpu.SemaphoreType.DMA,
            pltpu.SemaphoreType.DMA,
            pltpu.SemaphoreType.DMA,
            pltpu.SemaphoreType.DMA,
        ],
        compiler_params=pltpu.CompilerParams(use_tc_tiling_on_sc=False),
    )


@functools.cache
def _mesh():
    return plsc.VectorSubcoreMesh(core_axis_name="c", subcore_axis_name="s",
                                  num_cores=NC, num_subcores=NS)


@functools.cache
def _sc_kernels():
    return {
        "gdmg": _make_gather(1, N, 16, 1152),
        "gxd": _make_gxd_packed(768),
        "cnt": _make_scatter_add(16, False, 1152),
        "layer": _make_layer_sc(192),
        "gpair": _make_gpair(384),
    }


BEP = 1024
BN = 2048
GE_ = (E // 2) // BEP
GN_ = N // BN


def _full(shape):
    return pl.BlockSpec(shape, lambda i: tuple(0 for _ in shape))


def _rows(blk, d):
    return pl.BlockSpec((blk, d), lambda i: (i, 0))


def _espec(d):
    return pl.BlockSpec((BEP, d), lambda i: (jnp.minimum(i, GE_ - 1), 0))


def _nspec(d):
    return pl.BlockSpec(
        (BN, d), lambda i: (jnp.clip(i - GE_, 0, GN_ - 1), 0))


def _edge_init_math(xdr, xdc, w1, b1, w2, b2):
    eps = 1e-8
    a = jnp.transpose(xdr)
    c = jnp.transpose(xdc)
    sx0, sx1, dg0, dg1 = a[0:1, :], a[1:2, :], a[2:3, :], a[3:4, :]
    dx0, dx1 = c[0:1, :], c[1:2, :]
    v0 = sx0 - dx0
    v1 = sx1 - dx1
    l2r = v0 * v0 + v1 * v1
    el = jnp.sqrt(l2r + eps)
    l2 = jnp.maximum(l2r, eps)
    t = jnp.clip(((dg0 - sx0) * (dx0 - sx0) + (dg1 - sx1) * (dx1 - sx1)) / l2,
                 0.0, 1.0)
    p0 = sx0 + t * (dx0 - sx0)
    p1 = sx1 + t * (dx1 - sx1)
    dfd = jnp.sqrt((dg0 - p0) ** 2 + (dg1 - p1) ** 2 + eps)
    dtx = jnp.sqrt((sx0 - dg0) ** 2 + (sx1 - dg1) ** 2 + eps)
    drx = jnp.sqrt((dx0 - dg0) ** 2 + (dx1 - dg1) ** 2 + eps)
    phys_t = jnp.concatenate([v0, v1, el, dfd, dtx, drx], axis=0)
    pre = lax.dot_general(phys_t, w1[...], (((0,), (0,)), ((), ())),
                          preferred_element_type=jnp.float32) + b1[...]
    hid = jnp.maximum(pre, 0.0)
    return jnp.dot(hid, w2[...], preferred_element_type=jnp.float32) + b2[...]


def _init_body(xdp, xd, w1, b1, w2, b2, wn, bn, out_he2, out_hn):
    pid = pl.program_id(0)

    @pl.when(pid < GE_)
    def _():
        he_e = _edge_init_math(xdp[:, 0:16], xdp[:, 32:48], w1, b1, w2, b2)
        he_o = _edge_init_math(xdp[:, 16:32], xdp[:, 48:64], w1, b1, w2, b2)
        out_he2[...] = jnp.concatenate([he_e, he_o], axis=1)

    @pl.when(pid >= GE_)
    def _():
        out_hn[...] = (xd[:, 0:1] * wn[0:1, :] + xd[:, 1:2] * wn[1:2, :]
                       + bn[...])


def _tc_init(xdp, xd, w1, b1, w2, b2, wn, bn):
    return pl.pallas_call(
        _init_body,
        grid=(GE_ + GN_,),
        in_specs=[
            _espec(2 * H), _nspec(16),
            _full((6, H)), _full((1, H)), _full((H, H)), _full((1, H)),
            _full((2, H)), _full((1, H)),
        ],
        out_specs=[_espec(2 * H), _nspec(H)],
        out_shape=[jax.ShapeDtypeStruct((E // 2, 2 * H), jnp.float32),
                   jax.ShapeDtypeStruct((N, H), jnp.float32)],
    )(xdp, xd, w1, b1, w2, b2, wn, bn)


def _layer_body(gr2, gc2, he2, hn, ag, cnt, w1a, w1b, w1c, b1, w2, b2,
                nw1a, nw1b, nb1, nw2, nb2, out_he2, out_hn):
    pid = pl.program_id(0)

    @pl.when(pid < GE_)
    def _():
        def upd(g_r, g_c, h):
            pre = (jnp.dot(g_r, w1a[...], preferred_element_type=jnp.float32)
                   + jnp.dot(g_c, w1b[...], preferred_element_type=jnp.float32)
                   + jnp.dot(h, w1c[...], preferred_element_type=jnp.float32)
                   + b1[...])
            hid = jnp.maximum(pre, 0.0)
            return h + jnp.dot(hid, w2[...],
                               preferred_element_type=jnp.float32) + b2[...]

        even = upd(gr2[:, 0:H], gc2[:, 0:H], he2[:, 0:H])
        odd = upd(gr2[:, H:2 * H], gc2[:, H:2 * H], he2[:, H:2 * H])
        out_he2[...] = jnp.concatenate([even, odd], axis=1)

    @pl.when(pid >= GE_)
    def _():
        aggr = ag[...] / jnp.maximum(cnt[:, 0:1], 1.0)
        pre = (jnp.dot(hn[...], nw1a[...], preferred_element_type=jnp.float32)
               + jnp.dot(aggr, nw1b[...], preferred_element_type=jnp.float32)
               + nb1[...])
        hid = jnp.maximum(pre, 0.0)
        out_hn[...] = hn[...] + jnp.dot(
            hid, nw2[...], preferred_element_type=jnp.float32) + nb2[...]


def _final_body(gr2, gc2, he2, w1a, w1b, w1c, b1, w2, b2,
                wd1, bd1, wd2t, bd2, out_pred):
    def upd(g_r, g_c, h):
        pre = (jnp.dot(g_r, w1a[...], preferred_element_type=jnp.float32)
               + jnp.dot(g_c, w1b[...], preferred_element_type=jnp.float32)
               + jnp.dot(h, w1c[...], preferred_element_type=jnp.float32)
               + b1[...])
        hid = jnp.maximum(pre, 0.0)
        return h + jnp.dot(hid, w2[...],
                           preferred_element_type=jnp.float32) + b2[...]

    def head(x):
        hid = jnp.maximum(
            jnp.dot(x, wd1[...], preferred_element_type=jnp.float32)
            + bd1[...], 0.0)
        logit = jnp.sum(hid * wd2t[...], axis=1, keepdims=True) + bd2[...]
        return 1.0 / (1.0 + jnp.exp(-logit))

    even = upd(gr2[:, 0:H], gc2[:, 0:H], he2[:, 0:H])
    odd = upd(gr2[:, H:2 * H], gc2[:, H:2 * H], he2[:, H:2 * H])
    out_pred[...] = 0.5 * (head(even) + head(odd))


def _tc_final(gr2, gc2, he2, w1a, w1b, w1c, b1, w2, b2, wd1, bd1, wd2t, bd2):
    return pl.pallas_call(
        _final_body,
        grid=(GE_,),
        in_specs=[
            _rows(BEP, 2 * H), _rows(BEP, 2 * H), _rows(BEP, 2 * H),
            _full((H, H)), _full((H, H)), _full((H, H)), _full((1, H)),
            _full((H, H)), _full((1, H)),
            _full((H, H // 2)), _full((1, H // 2)), _full((1, H // 2)),
            _full((1, 1)),
        ],
        out_specs=_rows(BEP, 1),
        out_shape=jax.ShapeDtypeStruct((E // 2, 1), jnp.float32),
    )(gr2, gc2, he2, w1a, w1b, w1c, b1, w2, b2, wd1, bd1, wd2t, bd2)


def _tc_layer(gr2, gc2, he2, hn, ag, cnt, w1a, w1b, w1c, b1, w2, b2,
              nw1a, nw1b, nb1, nw2, nb2):
    return pl.pallas_call(
        _layer_body,
        grid=(GE_ + GN_,),
        in_specs=[
            _espec(2 * H), _espec(2 * H), _espec(2 * H),
            _nspec(H), _nspec(H), _nspec(16),
            _full((H, H)), _full((H, H)), _full((H, H)), _full((1, H)),
            _full((H, H)), _full((1, H)),
            _full((H, H)), _full((H, H)), _full((1, H)),
            _full((H, H)), _full((1, H)),
        ],
        out_specs=[_espec(2 * H), _nspec(H)],
        out_shape=[jax.ShapeDtypeStruct((E // 2, 2 * H), jnp.float32),
                   jax.ShapeDtypeStruct((N, H), jnp.float32)],
    )(gr2, gc2, he2, hn, ag, cnt, w1a, w1b, w1c, b1, w2, b2,
      nw1a, nw1b, nb1, nw2, nb2)


def _decoder_body(he2, wd1, bd1, wd2t, bd2, out):
    def head(x):
        hid = jnp.maximum(
            jnp.dot(x, wd1[...], preferred_element_type=jnp.float32) + bd1[...],
            0.0)
        logit = jnp.sum(hid * wd2t[...], axis=1, keepdims=True) + bd2[...]
        return 1.0 / (1.0 + jnp.exp(-logit))

    out[...] = 0.5 * (head(he2[:, 0:H]) + head(he2[:, H:2 * H]))


def _tc_decoder(he2, wd1, bd1, wd2t, bd2, blk=2048):
    return pl.pallas_call(
        _decoder_body,
        grid=(E // 2 // blk,),
        in_specs=[
            _rows(blk, 2 * H),
            _full((H, H // 2)), _full((1, H // 2)), _full((1, H // 2)),
            _full((1, 1)),
        ],
        out_specs=_rows(blk, 1),
        out_shape=jax.ShapeDtypeStruct((E // 2, 1), jnp.float32),
    )(he2, wd1, bd1, wd2t, bd2)


def kernel(x, edge_index, batch, damage_locs, W_ne, b_ne, W_ee1, b_ee1,
           W_ee2, b_ee2, W_em1, b_em1, W_em2, b_em2, W_nm1, b_nm1, W_nm2,
           b_nm2, W_d1, b_d1, W_d2, b_d2):
    row = edge_index[0]
    col = edge_index[1]
    row_e, row_o = row[0::2], row[1::2]
    col_e, col_o = col[0::2], col[1::2]
    sc = _sc_kernels()

    dmg_pad = jnp.pad(damage_locs, ((0, 0), (0, 14)))
    dmg_node, = sc["gdmg"](dmg_pad, batch)
    xd = jnp.concatenate(
        [x, dmg_node[:, :2], jnp.zeros((N, 12), jnp.float32)], axis=1)

    xdp = sc["gxd"](xd, row_e, row_o, col_e, col_o)
    h_e2, h_n = _tc_init(xdp, xd, W_ee1, b_ee1.reshape(1, H), W_ee2,
                         b_ee2.reshape(1, H), W_ne, b_ne.reshape(1, H))
    cnt = sc["cnt"](col)

    for l in range(L - 1):
        gr2, gc2, aggr = sc["layer"](h_n, row_e, row_o, col_e, col_o)
        h_e2, h_n = _tc_layer(
            gr2, gc2, h_e2, h_n, aggr, cnt,
            W_em1[l, 0:H], W_em1[l, H:2 * H], W_em1[l, 2 * H:3 * H],
            b_em1[l].reshape(1, H), W_em2[l], b_em2[l].reshape(1, H),
            W_nm1[l, 0:H], W_nm1[l, H:2 * H], b_nm1[l].reshape(1, H),
            W_nm2[l], b_nm2[l].reshape(1, H))

    gr2, gc2 = sc["gpair"](h_n, row_e, row_o, col_e, col_o)
    lf = L - 1
    pred2 = _tc_final(
        gr2, gc2, h_e2,
        W_em1[lf, 0:H], W_em1[lf, H:2 * H], W_em1[lf, 2 * H:3 * H],
        b_em1[lf].reshape(1, H), W_em2[lf], b_em2[lf].reshape(1, H),
        W_d1, b_d1.reshape(1, H // 2), W_d2.reshape(1, H // 2),
        b_d2.reshape(1, 1))
    return pred2.reshape(NB, PAIRS)
